# Initial kernel scaffold; baseline (speedup 1.0000x reference)
import jax
import jax.numpy as jnp
from jax.experimental import pallas as pl


def kernel(x, edge_index, W, b, gamma, beta):
    raise NotImplementedError("write your pallas kernel here")


# speedup vs baseline: 21.5508x; 21.5508x over previous
import jax
import jax.numpy as jnp
from jax import lax
from jax.experimental import pallas as pl
from jax.experimental.pallas import tpu as pltpu
from jax.experimental.pallas import tpu_sc as plsc

N = 10000
E = 320000
D = 128
NC, NS = 2, 16
NW = NC * NS
EPW = E // NW
C = 125
NCHUNK = EPW // C
RPT = N // NS
RC = RPT // C

_mesh = plsc.VectorSubcoreMesh(core_axis_name="c", subcore_axis_name="s")
_sc_params = pltpu.CompilerParams(use_tc_tiling_on_sc=False)


def _hist_body(dst_hbm, out_hbm, didx, ones_v, zrow_v, acc):
    c = lax.axis_index("c")
    s = lax.axis_index("s")

    def fill(j, _):
        ones_v[j] = jnp.full((16,), 1.0, jnp.float32)
        zrow_v[j] = jnp.zeros((16,), jnp.float32)
        return 0

    lax.fori_loop(0, C, fill, 0)

    def zslab(k, _):
        pltpu.sync_copy(zrow_v, acc.at[pl.ds(s * RPT + k * C, C)])
        return 0

    lax.fori_loop(0, RC, zslab, 0)
    plsc.subcore_barrier()

    def chunk(j, _):
        pltpu.sync_copy(dst_hbm.at[c, s, j], didx.at[0])
        pltpu.sync_copy(ones_v, acc.at[didx.at[0]], add=True)
        return 0

    lax.fori_loop(0, NCHUNK, chunk, 0)
    plsc.subcore_barrier()

    def cpout(k, _):
        r0 = s * RPT + k * C
        pltpu.sync_copy(acc.at[pl.ds(r0, C)], out_hbm.at[c, pl.ds(r0, C)])
        return 0

    lax.fori_loop(0, RC, cpout, 0)


_hist = pl.kernel(
    _hist_body,
    out_type=jax.ShapeDtypeStruct((NC, N, 16), jnp.float32),
    mesh=_mesh,
    compiler_params=_sc_params,
    scratch_types=[
        pltpu.VMEM((1, C), jnp.int32),
        pltpu.VMEM((C, 16), jnp.float32),
        pltpu.VMEM((C, 16), jnp.float32),
        pltpu.VMEM_SHARED((N, 16), jnp.float32),
    ],
)


def _scatter_body(y_hbm, src_hbm, dst_hbm, out_hbm, sidx, didx, rows, acc, sem):
    c = lax.axis_index("c")
    s = lax.axis_index("s")

    def zfill(j, _):
        for k in range(D // 16):
            rows[j, pl.ds(k * 16, 16)] = jnp.zeros((16,), jnp.float32)
        return 0

    lax.fori_loop(0, C, zfill, 0)

    def zslab(k, _):
        pltpu.sync_copy(rows, acc.at[pl.ds(s * RPT + k * C, C)])
        return 0

    lax.fori_loop(0, RC, zslab, 0)
    plsc.subcore_barrier()

    def chunk(j, _):
        pltpu.sync_copy(src_hbm.at[c, s, j], sidx.at[0])
        pltpu.sync_copy(dst_hbm.at[c, s, j], didx.at[0])
        pltpu.async_copy(y_hbm.at[sidx.at[0]], rows, sem).wait()
        pltpu.sync_copy(rows, acc.at[didx.at[0]], add=True)
        return 0

    lax.fori_loop(0, NCHUNK, chunk, 0)
    plsc.subcore_barrier()

    def cpout(k, _):
        r0 = s * RPT + k * C
        pltpu.sync_copy(acc.at[pl.ds(r0, C)], out_hbm.at[c, pl.ds(r0, C)])
        return 0

    lax.fori_loop(0, RC, cpout, 0)


_scatter = pl.kernel(
    _scatter_body,
    out_type=jax.ShapeDtypeStruct((NC, N, D), jnp.float32),
    mesh=_mesh,
    compiler_params=_sc_params,
    scratch_types=[
        pltpu.VMEM((1, C), jnp.int32),
        pltpu.VMEM((1, C), jnp.int32),
        pltpu.VMEM((C, D), jnp.float32),
        pltpu.VMEM_SHARED((N, D), jnp.float32),
        pltpu.SemaphoreType.DMA,
    ],
)


def _prescale_body(x_ref, w_ref, h_ref, y_ref, dinv_ref):
    deg = 1.0 + h_ref[0, :, 0] + h_ref[1, :, 0]
    dinv = lax.rsqrt(deg)
    xw = jnp.dot(x_ref[...], w_ref[...], preferred_element_type=jnp.float32)
    y_ref[...] = xw * dinv[:, None]
    dinv_ref[...] = dinv


_prescale = pl.pallas_call(
    _prescale_body,
    out_shape=(
        jax.ShapeDtypeStruct((N, D), jnp.float32),
        jax.ShapeDtypeStruct((N,), jnp.float32),
    ),
)


def _bn_body(p_ref, y_ref, dinv_ref, b_ref, g_ref, bt_ref, o_ref):
    tot = p_ref[0] + p_ref[1] + y_ref[...]
    pre = tot * dinv_ref[...][:, None] + b_ref[...][None, :]
    mean = jnp.mean(pre, axis=0)
    cen = pre - mean[None, :]
    var = jnp.mean(cen * cen, axis=0)
    o = cen * lax.rsqrt(var + 1e-5)[None, :] * g_ref[...][None, :] + bt_ref[...][None, :]
    o_ref[...] = jnp.where(o >= 0, o, 0.01 * o)


_bn = pl.pallas_call(
    _bn_body,
    out_shape=jax.ShapeDtypeStruct((N, D), jnp.float32),
)


def kernel(x, edge_index, W, b, gamma, beta):
    ei = edge_index.astype(jnp.int32)
    src = ei[0].reshape(NC, NS, NCHUNK, C)
    dst = ei[1].reshape(NC, NS, NCHUNK, C)
    hist = _hist(dst)
    y, dinv = _prescale(x, W, hist)
    p = _scatter(y, src, dst)
    return _bn(p, y, dinv, b, gamma, beta)


# speedup vs baseline: 42.1055x; 1.9538x over previous
import jax
import jax.numpy as jnp
from jax import lax
from jax.experimental import pallas as pl
from jax.experimental.pallas import tpu as pltpu
from jax.experimental.pallas import tpu_sc as plsc

N = 10000
E = 320000
D = 128
NC, NS = 2, 16
NW = NC * NS
EPW = E // NW
C = 125
NCHUNK = EPW // C
RPT = N // NS
RC = RPT // C

_mesh = plsc.VectorSubcoreMesh(core_axis_name="c", subcore_axis_name="s")
_sc_params = pltpu.CompilerParams(use_tc_tiling_on_sc=False)


def _hist_body(dst_hbm, out_hbm, dstv, ones_v, zrow_v, acc, sem):
    c = lax.axis_index("c")
    s = lax.axis_index("s")

    def fill(j, _):
        ones_v[j] = jnp.full((16,), 1.0, jnp.float32)
        zrow_v[j] = jnp.zeros((16,), jnp.float32)
        return 0

    lax.fori_loop(0, C, fill, 0)

    pltpu.sync_copy(dst_hbm.at[c, s], dstv)

    def zslab(k, _):
        pltpu.sync_copy(zrow_v, acc.at[pl.ds(s * RPT + k * C, C)])
        return 0

    lax.fori_loop(0, RC, zslab, 0)
    plsc.subcore_barrier()

    G = 8

    def grp(g, _):
        for t in range(G):
            pltpu.async_copy(ones_v, acc.at[dstv.at[g * G + t]], sem, add=True)
        for t in range(G):
            pltpu.make_async_copy(ones_v, acc.at[pl.ds(0, C)], sem).wait()
        return 0

    lax.fori_loop(0, NCHUNK // G, grp, 0)
    plsc.subcore_barrier()

    def cpout(k, _):
        r0 = s * RPT + k * C
        pltpu.sync_copy(acc.at[pl.ds(r0, C)], out_hbm.at[c, pl.ds(r0, C)])
        return 0

    lax.fori_loop(0, RC, cpout, 0)


_hist = pl.kernel(
    _hist_body,
    out_type=jax.ShapeDtypeStruct((NC, N, 16), jnp.float32),
    mesh=_mesh,
    compiler_params=_sc_params,
    scratch_types=[
        pltpu.VMEM((NCHUNK, C), jnp.int32),
        pltpu.VMEM((C, 16), jnp.float32),
        pltpu.VMEM((C, 16), jnp.float32),
        pltpu.VMEM_SHARED((N, 16), jnp.float32),
        pltpu.SemaphoreType.DMA,
    ],
)


HALF = NCHUNK // 2
NPAIR = HALF // 2


def _scatter_body(y_hbm, src_hbm, dst_hbm, out_hbm, srcv, dstv, rows0, rows1, acc, sem0, sem1):
    c = lax.axis_index("c")
    s = lax.axis_index("s")

    def zfill(j, _):
        for k in range(D // 16):
            rows0[j, pl.ds(k * 16, 16)] = jnp.zeros((16,), jnp.float32)
        return 0

    lax.fori_loop(0, C, zfill, 0)

    def zslab(k, _):
        pltpu.sync_copy(rows0, acc.at[pl.ds(s * RPT + k * C, C)])
        return 0

    lax.fori_loop(0, RC, zslab, 0)
    plsc.subcore_barrier()

    for h in range(2):
        pltpu.sync_copy(src_hbm.at[c, s, pl.ds(h * HALF, HALF)], srcv)
        pltpu.sync_copy(dst_hbm.at[c, s, pl.ds(h * HALF, HALF)], dstv)
        pltpu.async_copy(y_hbm.at[srcv.at[0]], rows0, sem0)
        pltpu.async_copy(y_hbm.at[srcv.at[1]], rows1, sem1)

        def pair(jj, _):
            j0 = 2 * jj
            pltpu.make_async_copy(y_hbm.at[pl.ds(0, C)], rows0, sem0).wait()
            pltpu.sync_copy(rows0, acc.at[dstv.at[j0]], add=True)

            @pl.when(jj < NPAIR - 1)
            def _():
                pltpu.async_copy(y_hbm.at[srcv.at[j0 + 2]], rows0, sem0)

            pltpu.make_async_copy(y_hbm.at[pl.ds(0, C)], rows1, sem1).wait()
            pltpu.sync_copy(rows1, acc.at[dstv.at[j0 + 1]], add=True)

            @pl.when(jj < NPAIR - 1)
            def _():
                pltpu.async_copy(y_hbm.at[srcv.at[j0 + 3]], rows1, sem1)

            return 0

        lax.fori_loop(0, NPAIR, pair, 0)
    plsc.subcore_barrier()

    def cpout(k, _):
        r0 = s * RPT + k * C
        pltpu.sync_copy(acc.at[pl.ds(r0, C)], out_hbm.at[c, pl.ds(r0, C)])
        return 0

    lax.fori_loop(0, RC, cpout, 0)


_scatter = pl.kernel(
    _scatter_body,
    out_type=jax.ShapeDtypeStruct((NC, N, D), jnp.float32),
    mesh=_mesh,
    compiler_params=_sc_params,
    scratch_types=[
        pltpu.VMEM((HALF, C), jnp.int32),
        pltpu.VMEM((HALF, C), jnp.int32),
        pltpu.VMEM((C, D), jnp.float32),
        pltpu.VMEM((C, D), jnp.float32),
        pltpu.VMEM_SHARED((N, D), jnp.float32),
        pltpu.SemaphoreType.DMA,
        pltpu.SemaphoreType.DMA,
    ],
)


def _prescale_body(x_ref, w_ref, h_ref, y_ref, dinv_ref):
    deg = 1.0 + h_ref[0, :, 0] + h_ref[1, :, 0]
    dinv = lax.rsqrt(deg)
    xw = jnp.dot(x_ref[...], w_ref[...], preferred_element_type=jnp.float32)
    y_ref[...] = xw * dinv[:, None]
    dinv_ref[...] = dinv


_prescale = pl.pallas_call(
    _prescale_body,
    out_shape=(
        jax.ShapeDtypeStruct((N, D), jnp.float32),
        jax.ShapeDtypeStruct((N,), jnp.float32),
    ),
)


def _bn_body(p_ref, y_ref, dinv_ref, b_ref, g_ref, bt_ref, o_ref):
    tot = p_ref[0] + p_ref[1] + y_ref[...]
    pre = tot * dinv_ref[...][:, None] + b_ref[...][None, :]
    mean = jnp.mean(pre, axis=0)
    cen = pre - mean[None, :]
    var = jnp.mean(cen * cen, axis=0)
    o = cen * lax.rsqrt(var + 1e-5)[None, :] * g_ref[...][None, :] + bt_ref[...][None, :]
    o_ref[...] = jnp.where(o >= 0, o, 0.01 * o)


_bn = pl.pallas_call(
    _bn_body,
    out_shape=jax.ShapeDtypeStruct((N, D), jnp.float32),
)


def kernel(x, edge_index, W, b, gamma, beta):
    ei = edge_index.astype(jnp.int32)
    src = ei[0].reshape(NC, NS, NCHUNK, C)
    dst = ei[1].reshape(NC, NS, NCHUNK, C)
    hist = _hist(dst)
    y, dinv = _prescale(x, W, hist)
    p = _scatter(y, src, dst)
    return _bn(p, y, dinv, b, gamma, beta)


# speedup vs baseline: 43.1142x; 1.0240x over previous
import jax
import jax.numpy as jnp
from jax import lax
from jax.experimental import pallas as pl
from jax.experimental.pallas import tpu as pltpu
from jax.experimental.pallas import tpu_sc as plsc

N = 10000
E = 320000
D = 128
NC, NS = 2, 16
NW = NC * NS
EPW = E // NW
C = 125
NCHUNK = EPW // C
RPT = N // NS
RC = RPT // C

_mesh = plsc.VectorSubcoreMesh(core_axis_name="c", subcore_axis_name="s")
_sc_params = pltpu.CompilerParams(use_tc_tiling_on_sc=False)


def _hist_body(ei_hbm, out_hbm, dstv, ones_v, zrow_v, acc, sem):
    c = lax.axis_index("c")
    s = lax.axis_index("s")

    def fill(j, _):
        ones_v[j] = jnp.full((16,), 1.0, jnp.float32)
        zrow_v[j] = jnp.zeros((16,), jnp.float32)
        return 0

    lax.fori_loop(0, C, fill, 0)

    pltpu.sync_copy(ei_hbm.at[1, c, s], dstv)

    def zslab(k, _):
        pltpu.sync_copy(zrow_v, acc.at[pl.ds(s * RPT + k * C, C)])
        return 0

    lax.fori_loop(0, RC, zslab, 0)
    plsc.subcore_barrier()

    G = 8

    def grp(g, _):
        for t in range(G):
            pltpu.async_copy(ones_v, acc.at[dstv.at[g * G + t]], sem, add=True)
        for t in range(G):
            pltpu.make_async_copy(ones_v, acc.at[pl.ds(0, C)], sem).wait()
        return 0

    lax.fori_loop(0, NCHUNK // G, grp, 0)
    plsc.subcore_barrier()

    def cpout(k, _):
        r0 = s * RPT + k * C
        pltpu.sync_copy(acc.at[pl.ds(r0, C)], out_hbm.at[c, pl.ds(r0, C)])
        return 0

    lax.fori_loop(0, RC, cpout, 0)


_hist = pl.kernel(
    _hist_body,
    out_type=jax.ShapeDtypeStruct((NC, N, 16), jnp.float32),
    mesh=_mesh,
    compiler_params=_sc_params,
    scratch_types=[
        pltpu.VMEM((NCHUNK, C), jnp.int32),
        pltpu.VMEM((C, 16), jnp.float32),
        pltpu.VMEM((C, 16), jnp.float32),
        pltpu.VMEM_SHARED((N, 16), jnp.float32),
        pltpu.SemaphoreType.DMA,
    ],
)


HALF = NCHUNK // 2
NPAIR = HALF // 2


def _scatter_body(y_hbm, ei_hbm, out_hbm, srcv, dstv, rows0, rows1, acc, sem0, sem1):
    c = lax.axis_index("c")
    s = lax.axis_index("s")

    @pl.when(c == 0)
    def _():
        def yslab(k, _):
            r0 = s * RPT + k * C
            pltpu.sync_copy(y_hbm.at[pl.ds(r0, C)], rows0)
            pltpu.sync_copy(rows0, acc.at[pl.ds(r0, C)])
            return 0

        lax.fori_loop(0, RC, yslab, 0)

    @pl.when(c == 1)
    def _():
        def zfill(j, _):
            for k in range(D // 16):
                rows0[j, pl.ds(k * 16, 16)] = jnp.zeros((16,), jnp.float32)
            return 0

        lax.fori_loop(0, C, zfill, 0)

        def zslab(k, _):
            pltpu.sync_copy(rows0, acc.at[pl.ds(s * RPT + k * C, C)])
            return 0

        lax.fori_loop(0, RC, zslab, 0)

    plsc.subcore_barrier()

    for h in range(2):
        pltpu.sync_copy(ei_hbm.at[0, c, s, pl.ds(h * HALF, HALF)], srcv)
        pltpu.sync_copy(ei_hbm.at[1, c, s, pl.ds(h * HALF, HALF)], dstv)
        pltpu.async_copy(y_hbm.at[srcv.at[0]], rows0, sem0)
        pltpu.async_copy(y_hbm.at[srcv.at[1]], rows1, sem1)

        def pair(jj, _):
            j0 = 2 * jj
            pltpu.make_async_copy(y_hbm.at[pl.ds(0, C)], rows0, sem0).wait()
            pltpu.sync_copy(rows0, acc.at[dstv.at[j0]], add=True)

            @pl.when(jj < NPAIR - 1)
            def _():
                pltpu.async_copy(y_hbm.at[srcv.at[j0 + 2]], rows0, sem0)

            pltpu.make_async_copy(y_hbm.at[pl.ds(0, C)], rows1, sem1).wait()
            pltpu.sync_copy(rows1, acc.at[dstv.at[j0 + 1]], add=True)

            @pl.when(jj < NPAIR - 1)
            def _():
                pltpu.async_copy(y_hbm.at[srcv.at[j0 + 3]], rows1, sem1)

            return 0

        lax.fori_loop(0, NPAIR, pair, 0)
    plsc.subcore_barrier()

    def cpout(k, _):
        r0 = s * RPT + k * C
        pltpu.sync_copy(acc.at[pl.ds(r0, C)], out_hbm.at[c, pl.ds(r0, C)])
        return 0

    lax.fori_loop(0, RC, cpout, 0)


_scatter = pl.kernel(
    _scatter_body,
    out_type=jax.ShapeDtypeStruct((NC, N, D), jnp.float32),
    mesh=_mesh,
    compiler_params=_sc_params,
    scratch_types=[
        pltpu.VMEM((HALF, C), jnp.int32),
        pltpu.VMEM((HALF, C), jnp.int32),
        pltpu.VMEM((C, D), jnp.float32),
        pltpu.VMEM((C, D), jnp.float32),
        pltpu.VMEM_SHARED((N, D), jnp.float32),
        pltpu.SemaphoreType.DMA,
        pltpu.SemaphoreType.DMA,
    ],
)


def _prescale_body(x_ref, w_ref, h_ref, y_ref, dinv_ref):
    deg = 1.0 + h_ref[0, :, 0] + h_ref[1, :, 0]
    dinv = lax.rsqrt(deg)
    xw = jnp.dot(x_ref[...], w_ref[...], preferred_element_type=jnp.float32)
    y_ref[...] = xw * dinv[:, None]
    dinv_ref[...] = dinv


_prescale = pl.pallas_call(
    _prescale_body,
    out_shape=(
        jax.ShapeDtypeStruct((N, D), jnp.float32),
        jax.ShapeDtypeStruct((N,), jnp.float32),
    ),
)


def _bn_body(p_ref, dinv_ref, b_ref, g_ref, bt_ref, o_ref):
    tot = p_ref[0] + p_ref[1]
    pre = tot * dinv_ref[...][:, None] + b_ref[...][None, :]
    mean = jnp.mean(pre, axis=0)
    cen = pre - mean[None, :]
    var = jnp.mean(cen * cen, axis=0)
    o = cen * lax.rsqrt(var + 1e-5)[None, :] * g_ref[...][None, :] + bt_ref[...][None, :]
    o_ref[...] = jnp.where(o >= 0, o, 0.01 * o)


_bn = pl.pallas_call(
    _bn_body,
    out_shape=jax.ShapeDtypeStruct((N, D), jnp.float32),
)


def kernel(x, edge_index, W, b, gamma, beta):
    ei = jnp.reshape(edge_index.astype(jnp.int32), (2, NC, NS, NCHUNK, C))
    hist = _hist(ei)
    y, dinv = _prescale(x, W, hist)
    p = _scatter(y, ei)
    return _bn(p, dinv, b, gamma, beta)


# speedup vs baseline: 50.2204x; 1.1648x over previous
import jax
import jax.numpy as jnp
from jax import lax
from jax.experimental import pallas as pl
from jax.experimental.pallas import tpu as pltpu
from jax.experimental.pallas import tpu_sc as plsc

N = 10000
E = 320000
D = 128
NC, NS = 2, 16
NW = NC * NS
EPW = E // NW
C = 125
NCHUNK = EPW // C
RPT = N // NS
RC = RPT // C

_mesh = plsc.VectorSubcoreMesh(core_axis_name="c", subcore_axis_name="s")
_sc_params = pltpu.CompilerParams(use_tc_tiling_on_sc=False)


def _hist_body(ei_hbm, out_hbm, dstv, ones_v, zrow_v, acc, sem):
    c = lax.axis_index("c")
    s = lax.axis_index("s")

    def fill(j, _):
        ones_v[j] = jnp.full((16,), 1.0, jnp.float32)
        zrow_v[j] = jnp.zeros((16,), jnp.float32)
        return 0

    lax.fori_loop(0, C, fill, 0)

    pltpu.sync_copy(ei_hbm.at[1, c, s], dstv)

    def zslab(k, _):
        pltpu.sync_copy(zrow_v, acc.at[pl.ds(s * RPT + k * C, C)])
        return 0

    lax.fori_loop(0, RC, zslab, 0)
    plsc.subcore_barrier()

    G = 8

    def grp(g, _):
        for t in range(G):
            pltpu.async_copy(ones_v, acc.at[dstv.at[g * G + t]], sem, add=True)
        for t in range(G):
            pltpu.make_async_copy(ones_v, acc.at[pl.ds(0, C)], sem).wait()
        return 0

    lax.fori_loop(0, NCHUNK // G, grp, 0)
    plsc.subcore_barrier()

    def cpout(k, _):
        r0 = s * RPT + k * C
        pltpu.sync_copy(acc.at[pl.ds(r0, C)], out_hbm.at[c, pl.ds(r0, C)])
        return 0

    lax.fori_loop(0, RC, cpout, 0)


_hist = pl.kernel(
    _hist_body,
    out_type=jax.ShapeDtypeStruct((NC, N, 16), jnp.float32),
    mesh=_mesh,
    compiler_params=_sc_params,
    scratch_types=[
        pltpu.VMEM((NCHUNK, C), jnp.int32),
        pltpu.VMEM((C, 16), jnp.float32),
        pltpu.VMEM((C, 16), jnp.float32),
        pltpu.VMEM_SHARED((N, 16), jnp.float32),
        pltpu.SemaphoreType.DMA,
    ],
)


NBUF = 4
NGRP = NCHUNK // NBUF


def _scatter_body(y_hbm, ei_hbm, out_hbm, srcv, dstv, r0b, r1b, r2b, r3b, acc,
                  s0, s1, s2, s3):
    c = lax.axis_index("c")
    s = lax.axis_index("s")
    bufs = (r0b, r1b, r2b, r3b)
    sems = (s0, s1, s2, s3)

    @pl.when(c == 0)
    def _():
        def yslab(k, _):
            r0 = s * RPT + k * C
            pltpu.sync_copy(y_hbm.at[pl.ds(r0, C)], r0b)
            pltpu.sync_copy(r0b, acc.at[pl.ds(r0, C)])
            return 0

        lax.fori_loop(0, RC, yslab, 0)

    @pl.when(c == 1)
    def _():
        def zfill(j, _):
            for k in range(D // 32):
                r0b[j, pl.ds(k * 32, 32)] = jnp.zeros((32,), jnp.bfloat16)
            return 0

        lax.fori_loop(0, C, zfill, 0)

        def zslab(k, _):
            pltpu.sync_copy(r0b, acc.at[pl.ds(s * RPT + k * C, C)])
            return 0

        lax.fori_loop(0, RC, zslab, 0)

    pltpu.sync_copy(ei_hbm.at[0, c, s], srcv)
    pltpu.sync_copy(ei_hbm.at[1, c, s], dstv)
    plsc.subcore_barrier()

    for t in range(NBUF):
        pltpu.async_copy(y_hbm.at[srcv.at[t]], bufs[t], sems[t])

    def grp(jj, _):
        j0 = jj * NBUF
        for t in range(NBUF):
            pltpu.make_async_copy(y_hbm.at[pl.ds(0, C)], bufs[t], sems[t]).wait()
            pltpu.sync_copy(bufs[t], acc.at[dstv.at[j0 + t]], add=True)

            @pl.when(jj < NGRP - 1)
            def _():
                pltpu.async_copy(y_hbm.at[srcv.at[j0 + t + NBUF]], bufs[t], sems[t])

        return 0

    lax.fori_loop(0, NGRP, grp, 0)
    plsc.subcore_barrier()

    def cpout(k, _):
        r0 = s * RPT + k * C
        pltpu.sync_copy(acc.at[pl.ds(r0, C)], out_hbm.at[c, pl.ds(r0, C)])
        return 0

    lax.fori_loop(0, RC, cpout, 0)


_scatter = pl.kernel(
    _scatter_body,
    out_type=jax.ShapeDtypeStruct((NC, N, D), jnp.bfloat16),
    mesh=_mesh,
    compiler_params=_sc_params,
    scratch_types=[
        pltpu.VMEM((NCHUNK, C), jnp.int32),
        pltpu.VMEM((NCHUNK, C), jnp.int32),
        pltpu.VMEM((C, D), jnp.bfloat16),
        pltpu.VMEM((C, D), jnp.bfloat16),
        pltpu.VMEM((C, D), jnp.bfloat16),
        pltpu.VMEM((C, D), jnp.bfloat16),
        pltpu.VMEM_SHARED((N, D), jnp.bfloat16),
        pltpu.SemaphoreType.DMA,
        pltpu.SemaphoreType.DMA,
        pltpu.SemaphoreType.DMA,
        pltpu.SemaphoreType.DMA,
    ],
)


def _prescale_body(x_ref, w_ref, h_ref, y_ref, dinv_ref):
    deg = 1.0 + h_ref[0, :, 0] + h_ref[1, :, 0]
    dinv = lax.rsqrt(deg)
    xw = jnp.dot(x_ref[...], w_ref[...], preferred_element_type=jnp.float32)
    y_ref[...] = (xw * dinv[:, None]).astype(jnp.bfloat16)
    dinv_ref[...] = dinv


_prescale = pl.pallas_call(
    _prescale_body,
    out_shape=(
        jax.ShapeDtypeStruct((N, D), jnp.bfloat16),
        jax.ShapeDtypeStruct((N,), jnp.float32),
    ),
)


def _bn_body(p_ref, dinv_ref, b_ref, g_ref, bt_ref, o_ref):
    tot = p_ref[0].astype(jnp.float32) + p_ref[1].astype(jnp.float32)
    pre = tot * dinv_ref[...][:, None] + b_ref[...][None, :]
    mean = jnp.mean(pre, axis=0)
    cen = pre - mean[None, :]
    var = jnp.mean(cen * cen, axis=0)
    o = cen * lax.rsqrt(var + 1e-5)[None, :] * g_ref[...][None, :] + bt_ref[...][None, :]
    o_ref[...] = jnp.where(o >= 0, o, 0.01 * o)


_bn = pl.pallas_call(
    _bn_body,
    out_shape=jax.ShapeDtypeStruct((N, D), jnp.float32),
)


def kernel(x, edge_index, W, b, gamma, beta):
    ei = jnp.reshape(edge_index.astype(jnp.int32), (2, NC, NS, NCHUNK, C))
    hist = _hist(ei)
    y, dinv = _prescale(x, W, hist)
    p = _scatter(y, ei)
    return _bn(p, dinv, b, gamma, beta)


# speedup vs baseline: 55.1329x; 1.0978x over previous
import jax
import jax.numpy as jnp
from jax import lax
from jax.experimental import pallas as pl
from jax.experimental.pallas import tpu as pltpu
from jax.experimental.pallas import tpu_sc as plsc

N = 10000
E = 320000
D = 128
NC, NS = 2, 16
NW = NC * NS
EPW = E // NW
C = 125
NCHUNK = EPW // C
RPT = N // NS
RC = RPT // C
NPAD = 10240
HPT = NPAD // NS

_mesh = plsc.VectorSubcoreMesh(core_axis_name="c", subcore_axis_name="s")
_sc_params = pltpu.CompilerParams(use_tc_tiling_on_sc=False)


def _hist_body(ei_hbm, out_hbm, dstv, ones_v, zrow_v, slab_v, deg_v, acc, sem):
    c = lax.axis_index("c")
    s = lax.axis_index("s")

    def fill(j, _):
        ones_v[j] = jnp.full((16,), 1.0, jnp.float32)
        return 0

    lax.fori_loop(0, C, fill, 0)

    def zfill(j, _):
        zrow_v[j] = jnp.zeros((16,), jnp.float32)
        return 0

    lax.fori_loop(0, 128, zfill, 0)

    pltpu.sync_copy(ei_hbm.at[1, c, s], dstv)

    def zslab(k, _):
        pltpu.sync_copy(zrow_v, acc.at[pl.ds(s * HPT + k * 128, 128)])
        return 0

    lax.fori_loop(0, HPT // 128, zslab, 0)
    plsc.subcore_barrier()

    G = 8

    def grp(g, _):
        for t in range(G):
            pltpu.async_copy(ones_v, acc.at[dstv.at[g * G + t]], sem, add=True)
        for t in range(G):
            pltpu.make_async_copy(ones_v, acc.at[pl.ds(0, C)], sem).wait()
        return 0

    lax.fori_loop(0, NCHUNK // G, grp, 0)
    plsc.subcore_barrier()

    pltpu.sync_copy(acc.at[pl.ds(s * HPT, HPT)], slab_v)
    lane = lax.iota(jnp.int32, 16)

    def ext(g, _):
        v = jnp.zeros((16,), jnp.float32)
        for r in range(16):
            v = jnp.where(lane == r, slab_v[g * 16 + r], v)
        deg_v[pl.ds(g * 16, 16)] = v
        return 0

    lax.fori_loop(0, HPT // 16, ext, 0)
    pltpu.sync_copy(deg_v, out_hbm.at[pl.ds(c * NPAD + s * HPT, HPT)])


_hist = pl.kernel(
    _hist_body,
    out_type=jax.ShapeDtypeStruct((NC * NPAD,), jnp.float32),
    mesh=_mesh,
    compiler_params=_sc_params,
    scratch_types=[
        pltpu.VMEM((NCHUNK, C), jnp.int32),
        pltpu.VMEM((C, 16), jnp.float32),
        pltpu.VMEM((128, 16), jnp.float32),
        pltpu.VMEM((HPT, 16), jnp.float32),
        pltpu.VMEM((HPT,), jnp.float32),
        pltpu.VMEM_SHARED((NPAD, 16), jnp.float32),
        pltpu.SemaphoreType.DMA,
    ],
)


NBUF = 4
NGRP = NCHUNK // NBUF


def _scatter_body(y_hbm, ei_hbm, out_hbm, srcv, dstv, r0b, r1b, r2b, r3b, acc,
                  s0, s1, s2, s3):
    c = lax.axis_index("c")
    s = lax.axis_index("s")
    bufs = (r0b, r1b, r2b, r3b)
    sems = (s0, s1, s2, s3)

    @pl.when(c == 0)
    def _():
        def yslab(k, _):
            r0 = s * RPT + k * C
            pltpu.sync_copy(y_hbm.at[pl.ds(r0, C)], r0b)
            pltpu.sync_copy(r0b, acc.at[pl.ds(r0, C)])
            return 0

        lax.fori_loop(0, RC, yslab, 0)

    @pl.when(c == 1)
    def _():
        def zfill(j, _):
            for k in range(D // 32):
                r0b[j, pl.ds(k * 32, 32)] = jnp.zeros((32,), jnp.bfloat16)
            return 0

        lax.fori_loop(0, C, zfill, 0)

        def zslab(k, _):
            pltpu.sync_copy(r0b, acc.at[pl.ds(s * RPT + k * C, C)])
            return 0

        lax.fori_loop(0, RC, zslab, 0)

    pltpu.sync_copy(ei_hbm.at[0, c, s], srcv)
    pltpu.sync_copy(ei_hbm.at[1, c, s], dstv)
    plsc.subcore_barrier()

    for t in range(NBUF):
        pltpu.async_copy(y_hbm.at[srcv.at[t]], bufs[t], sems[t])

    def grp(jj, _):
        j0 = jj * NBUF
        for t in range(NBUF):
            pltpu.make_async_copy(y_hbm.at[pl.ds(0, C)], bufs[t], sems[t]).wait()
            pltpu.sync_copy(bufs[t], acc.at[dstv.at[j0 + t]], add=True)

            @pl.when(jj < NGRP - 1)
            def _():
                pltpu.async_copy(y_hbm.at[srcv.at[j0 + t + NBUF]], bufs[t], sems[t])

        return 0

    lax.fori_loop(0, NGRP, grp, 0)
    plsc.subcore_barrier()

    def cpout(k, _):
        r0 = s * RPT + k * C
        pltpu.sync_copy(acc.at[pl.ds(r0, C)], out_hbm.at[c, pl.ds(r0, C)])
        return 0

    lax.fori_loop(0, RC, cpout, 0)


_scatter = pl.kernel(
    _scatter_body,
    out_type=jax.ShapeDtypeStruct((NC, N, D), jnp.bfloat16),
    mesh=_mesh,
    compiler_params=_sc_params,
    scratch_types=[
        pltpu.VMEM((NCHUNK, C), jnp.int32),
        pltpu.VMEM((NCHUNK, C), jnp.int32),
        pltpu.VMEM((C, D), jnp.bfloat16),
        pltpu.VMEM((C, D), jnp.bfloat16),
        pltpu.VMEM((C, D), jnp.bfloat16),
        pltpu.VMEM((C, D), jnp.bfloat16),
        pltpu.VMEM_SHARED((N, D), jnp.bfloat16),
        pltpu.SemaphoreType.DMA,
        pltpu.SemaphoreType.DMA,
        pltpu.SemaphoreType.DMA,
        pltpu.SemaphoreType.DMA,
    ],
)


def _mm_body(x_ref, w_ref, xw_ref):
    xw_ref[...] = jnp.dot(x_ref[...], w_ref[...], preferred_element_type=jnp.float32)


_mm = pl.pallas_call(
    _mm_body,
    out_shape=jax.ShapeDtypeStruct((N, D), jnp.float32),
)


def _scale_body(xw_ref, h_ref, y_ref, dinv_ref):
    h = h_ref[...]
    deg = 1.0 + h[0:NPAD] + h[NPAD : 2 * NPAD]
    dinv = lax.rsqrt(deg)[0:N]
    y_ref[...] = (xw_ref[...] * dinv[:, None]).astype(jnp.bfloat16)
    dinv_ref[...] = dinv


_scale = pl.pallas_call(
    _scale_body,
    out_shape=(
        jax.ShapeDtypeStruct((N, D), jnp.bfloat16),
        jax.ShapeDtypeStruct((N,), jnp.float32),
    ),
)


def _bn_body(p_ref, dinv_ref, b_ref, g_ref, bt_ref, o_ref):
    tot = p_ref[0].astype(jnp.float32) + p_ref[1].astype(jnp.float32)
    pre = tot * dinv_ref[...][:, None] + b_ref[...][None, :]
    mean = jnp.mean(pre, axis=0)
    cen = pre - mean[None, :]
    var = jnp.mean(cen * cen, axis=0)
    o = cen * lax.rsqrt(var + 1e-5)[None, :] * g_ref[...][None, :] + bt_ref[...][None, :]
    o_ref[...] = jnp.where(o >= 0, o, 0.01 * o)


_bn = pl.pallas_call(
    _bn_body,
    out_shape=jax.ShapeDtypeStruct((N, D), jnp.float32),
)


def kernel(x, edge_index, W, b, gamma, beta):
    ei = jnp.reshape(edge_index.astype(jnp.int32), (2, NC, NS, NCHUNK, C))
    hist = _hist(ei)
    xw = _mm(x, W)
    y, dinv = _scale(xw, hist)
    p = _scatter(y, ei)
    return _bn(p, dinv, b, gamma, beta)
